# baseline (speedup 1.0000x reference)
import functools

import jax
import jax.numpy as jnp
from jax import lax
from jax.experimental import pallas as pl
from jax.experimental.pallas import tpu as pltpu
from jax.experimental.pallas import tpu_sc as plsc

N = 10000
D = 128
NPAD = 10240
NBLK = NPAD // 128
NC = 2
NS = 16
NW = NC * NS
LK = 64
GRP = 4
CHUNKS = 164
GROUPS = CHUNKS // GRP
EPAD = NW * CHUNKS * LK
ROWS_PER_TILE = NPAD // NS


def _mesh():
    return plsc.VectorSubcoreMesh(core_axis_name="c", subcore_axis_name="s")


def _split_packed(pk, k, ri, ci):
    for m in range(LK // 16):
        sl = pl.ds(m * 16, 16)
        p = pk[k, sl]
        ri[sl] = p & 0x7FFF
        ci[sl] = lax.shift_right_logical(p, 15)


@functools.partial(
    pl.kernel,
    out_type=jax.ShapeDtypeStruct((NC * NPAD,), jnp.float32),
    mesh=_mesh(),
    scratch_types=[
        pltpu.VMEM((GRP, LK), jnp.int32),
        pltpu.VMEM((LK,), jnp.int32),
        pltpu.VMEM((LK,), jnp.int32),
        pltpu.VMEM((LK,), jnp.float32),
        pltpu.VMEM((ROWS_PER_TILE,), jnp.float32),
        pltpu.VMEM_SHARED((NPAD,), jnp.float32),
    ],
)
def _deg(pkb_hbm, zeros1_hbm, out_hbm, pk_v, ri_v, ci_v, ones_v, rb_v, deg_sh):
    c = lax.axis_index("c")
    s = lax.axis_index("s")
    wid = c * NS + s
    base = s * ROWS_PER_TILE
    pltpu.sync_copy(zeros1_hbm.at[pl.ds(base, ROWS_PER_TILE)],
                    deg_sh.at[pl.ds(base, ROWS_PER_TILE)])
    for i in range(LK // 16):
        ones_v[pl.ds(i * 16, 16)] = jnp.full((16,), 1.0, jnp.float32)
    plsc.subcore_barrier()

    def body(jj, carry):
        pltpu.sync_copy(pkb_hbm.at[pl.ds((jj * NW + wid) * GRP, GRP)], pk_v)
        for k in range(GRP):
            _split_packed(pk_v, k, ri_v, ci_v)
            pltpu.sync_copy(ones_v, deg_sh.at[ci_v], add=True)
        return carry

    lax.fori_loop(0, GROUPS, body, 0)
    plsc.subcore_barrier()
    pltpu.sync_copy(deg_sh.at[pl.ds(base, ROWS_PER_TILE)], rb_v)
    pltpu.sync_copy(rb_v, out_hbm.at[pl.ds(c * NPAD + base, ROWS_PER_TILE)])


@functools.partial(
    pl.kernel,
    out_type=jax.ShapeDtypeStruct((NC * NPAD, D), jnp.float32),
    mesh=_mesh(),
    scratch_types=[
        pltpu.VMEM((GRP, LK), jnp.int32),
        [pltpu.VMEM((LK,), jnp.int32) for _ in range(GRP)],
        [pltpu.VMEM((LK,), jnp.int32) for _ in range(GRP)],
        [pltpu.VMEM((LK, D), jnp.float32) for _ in range(GRP)],
        [pltpu.SemaphoreType.DMA for _ in range(GRP)],
        pltpu.VMEM_SHARED((NPAD, D), jnp.float32),
    ],
)
def _agg(y_hbm, pkb_hbm, zeros2_hbm, out_hbm,
         pk_v, ri, ci, buf, sg, acc_sh):
    c = lax.axis_index("c")
    s = lax.axis_index("s")
    wid = c * NS + s
    base = s * ROWS_PER_TILE
    pltpu.sync_copy(zeros2_hbm.at[pl.ds(base, ROWS_PER_TILE)],
                    acc_sh.at[pl.ds(base, ROWS_PER_TILE)])
    plsc.subcore_barrier()

    def body(jj, carry):
        pltpu.sync_copy(pkb_hbm.at[pl.ds((jj * NW + wid) * GRP, GRP)], pk_v)
        cps = []
        for k in range(GRP):
            _split_packed(pk_v, k, ri[k], ci[k])
            cps.append(pltpu.async_copy(y_hbm.at[ri[k]], buf[k], sg[k]))
        for k in range(GRP):
            cps[k].wait()
            pltpu.sync_copy(buf[k], acc_sh.at[ci[k]], add=True)
        return carry

    lax.fori_loop(0, GROUPS, body, 0)
    plsc.subcore_barrier()
    for k in range(ROWS_PER_TILE // LK):
        off = base + k * LK
        pltpu.sync_copy(acc_sh.at[pl.ds(off, LK)], buf[k % GRP])
        pltpu.sync_copy(buf[k % GRP], out_hbm.at[pl.ds(c * NPAD + off, LK)])


def _dinv_diag(d0_b, d1_b):
    dinv = lax.rsqrt(d0_b[0] + d1_b[0])
    ir = lax.broadcasted_iota(jnp.int32, (128, 128), 0)
    ic = lax.broadcasted_iota(jnp.int32, (128, 128), 1)
    return jnp.where(ir == ic, jnp.broadcast_to(dinv, (128, 128)), 0.0)


def _tc1_body(x_b, w_b, d0_b, d1_b, y_b):
    dm = _dinv_diag(d0_b, d1_b)
    xw = jnp.dot(x_b[...], w_b[...], preferred_element_type=jnp.float32)
    y_b[...] = jnp.dot(dm, xw, preferred_element_type=jnp.float32)


def _tc2_body(a0_b, a1_b, d0_b, d1_b, b1_b, w_b, y_b):
    dm = _dinv_diag(d0_b, d1_b)
    agg = a0_b[...] + a1_b[...]
    h = jnp.dot(dm, agg, preferred_element_type=jnp.float32) + b1_b[...]
    h = jnp.maximum(h, 0.0)
    hw = jnp.dot(h, w_b[...], preferred_element_type=jnp.float32)
    y_b[...] = jnp.dot(dm, hw, preferred_element_type=jnp.float32)


def _tc3_body(a0_b, a1_b, d0_b, d1_b, b2_b, y_b):
    dm = _dinv_diag(d0_b, d1_b)
    agg = a0_b[...] + a1_b[...]
    y_b[...] = jnp.dot(dm, agg, preferred_element_type=jnp.float32) + b2_b[...]


_ROWBLK = pl.BlockSpec((128, D), lambda i: (i, 0))
_FULLW = pl.BlockSpec((D, D), lambda i: (0, 0))
_DEGBLK = pl.BlockSpec((1, 1, 128), lambda i: (i, 0, 0))
_BIAS = pl.BlockSpec((1, D), lambda i: (0, 0))
_OUTBLK = jax.ShapeDtypeStruct((NPAD, D), jnp.float32)

_tc1 = pl.pallas_call(
    _tc1_body, grid=(NBLK,),
    in_specs=[_ROWBLK, _FULLW, _DEGBLK, _DEGBLK],
    out_specs=_ROWBLK, out_shape=_OUTBLK)

_tc2 = pl.pallas_call(
    _tc2_body, grid=(NBLK,),
    in_specs=[_ROWBLK, _ROWBLK, _DEGBLK, _DEGBLK, _BIAS, _FULLW],
    out_specs=_ROWBLK, out_shape=_OUTBLK)

_tc3 = pl.pallas_call(
    _tc3_body, grid=(NBLK,),
    in_specs=[_ROWBLK, _ROWBLK, _DEGBLK, _DEGBLK, _BIAS],
    out_specs=_ROWBLK, out_shape=_OUTBLK)


@jax.jit
def _run(x, edge_index, W1, b1, W2, b2):
    row = edge_index[0].astype(jnp.int32)
    col = edge_index[1].astype(jnp.int32)
    loop = jnp.arange(N, dtype=jnp.int32)
    npad_e = EPAD - (row.shape[0] + N)
    row = jnp.concatenate([row, loop, jnp.zeros((npad_e,), jnp.int32)])
    pad_col = N + (jnp.arange(npad_e, dtype=jnp.int32) % (NPAD - N))
    col = jnp.concatenate([col, loop, pad_col])
    packed = (row | (col << 15)).reshape(NW * CHUNKS, LK)

    xp = jnp.zeros((NPAD, D), x.dtype).at[:N].set(x)
    z1 = jnp.zeros((NPAD,), jnp.float32)
    z2 = jnp.zeros((NPAD, D), jnp.float32)

    deg = _deg(packed, z1).reshape(NC, NBLK, 1, 128)
    d0, d1 = deg[0], deg[1]

    y1 = _tc1(xp, W1, d0, d1)
    a = _agg(y1, packed, z2).reshape(NC, NPAD, D)
    y2 = _tc2(a[0], a[1], d0, d1, b1.reshape(1, D), W2)
    a = _agg(y2, packed, z2).reshape(NC, NPAD, D)
    out = _tc3(a[0], a[1], d0, d1, b2.reshape(1, D))
    return out[:N]


def kernel(x, edge_index, W1, b1, W2, b2):
    return _run(x, edge_index, W1, b1, W2, b2)


# speedup vs baseline: 1.1281x; 1.1281x over previous
import functools

import jax
import jax.numpy as jnp
from jax import lax
from jax.experimental import pallas as pl
from jax.experimental.pallas import tpu as pltpu
from jax.experimental.pallas import tpu_sc as plsc

N = 10000
D = 128
NPAD = 10240
NBLK = NPAD // 128
NC = 2
NS = 16
NW = NC * NS
LK = 128
GRP = 2
CHUNKS = 82
GROUPS = CHUNKS // GRP
EPAD = NW * CHUNKS * LK
ROWS_PER_TILE = NPAD // NS


def _mesh():
    return plsc.VectorSubcoreMesh(core_axis_name="c", subcore_axis_name="s")


def _split_packed(pk, k, ri, ci):
    for m in range(LK // 16):
        sl = pl.ds(m * 16, 16)
        p = pk[k, sl]
        ri[sl] = p & 0x7FFF
        ci[sl] = lax.shift_right_logical(p, 15)


@functools.partial(
    pl.kernel,
    out_type=jax.ShapeDtypeStruct((NC * NPAD,), jnp.float32),
    mesh=_mesh(),
    scratch_types=[
        pltpu.VMEM((GRP, LK), jnp.int32),
        pltpu.VMEM((LK,), jnp.int32),
        pltpu.VMEM((LK,), jnp.int32),
        pltpu.VMEM((LK,), jnp.float32),
        pltpu.VMEM((ROWS_PER_TILE,), jnp.float32),
        pltpu.VMEM_SHARED((NPAD,), jnp.float32),
    ],
)
def _deg(pkb_hbm, zeros1_hbm, out_hbm, pk_v, ri_v, ci_v, ones_v, rb_v, deg_sh):
    c = lax.axis_index("c")
    s = lax.axis_index("s")
    wid = c * NS + s
    base = s * ROWS_PER_TILE
    pltpu.sync_copy(zeros1_hbm.at[pl.ds(base, ROWS_PER_TILE)],
                    deg_sh.at[pl.ds(base, ROWS_PER_TILE)])
    for i in range(LK // 16):
        ones_v[pl.ds(i * 16, 16)] = jnp.full((16,), 1.0, jnp.float32)
    plsc.subcore_barrier()

    def body(jj, carry):
        pltpu.sync_copy(pkb_hbm.at[pl.ds((jj * NW + wid) * GRP, GRP)], pk_v)
        for k in range(GRP):
            _split_packed(pk_v, k, ri_v, ci_v)
            pltpu.sync_copy(ones_v, deg_sh.at[ci_v], add=True)
        return carry

    lax.fori_loop(0, GROUPS, body, 0)
    plsc.subcore_barrier()
    pltpu.sync_copy(deg_sh.at[pl.ds(base, ROWS_PER_TILE)], rb_v)
    pltpu.sync_copy(rb_v, out_hbm.at[pl.ds(c * NPAD + base, ROWS_PER_TILE)])


@functools.partial(
    pl.kernel,
    out_type=jax.ShapeDtypeStruct((NC * NPAD, D), jnp.float32),
    mesh=_mesh(),
    scratch_types=[
        pltpu.VMEM((GRP, LK), jnp.int32),
        [pltpu.VMEM((LK,), jnp.int32) for _ in range(GRP)],
        [pltpu.VMEM((LK,), jnp.int32) for _ in range(GRP)],
        [pltpu.VMEM((LK, D), jnp.float32) for _ in range(GRP)],
        [pltpu.SemaphoreType.DMA for _ in range(GRP)],
        pltpu.VMEM_SHARED((NPAD, D), jnp.float32),
    ],
)
def _agg(y_hbm, pkb_hbm, zeros2_hbm, out_hbm,
         pk_v, ri, ci, buf, sg, acc_sh):
    c = lax.axis_index("c")
    s = lax.axis_index("s")
    wid = c * NS + s
    base = s * ROWS_PER_TILE
    pltpu.sync_copy(zeros2_hbm.at[pl.ds(base, ROWS_PER_TILE)],
                    acc_sh.at[pl.ds(base, ROWS_PER_TILE)])
    plsc.subcore_barrier()

    def body(jj, carry):
        pltpu.sync_copy(pkb_hbm.at[pl.ds((jj * NW + wid) * GRP, GRP)], pk_v)
        cps = []
        for k in range(GRP):
            _split_packed(pk_v, k, ri[k], ci[k])
            cps.append(pltpu.async_copy(y_hbm.at[ri[k]], buf[k], sg[k]))
        for k in range(GRP):
            cps[k].wait()
            pltpu.sync_copy(buf[k], acc_sh.at[ci[k]], add=True)
        return carry

    lax.fori_loop(0, GROUPS, body, 0)
    plsc.subcore_barrier()
    for k in range(ROWS_PER_TILE // LK):
        off = base + k * LK
        pltpu.sync_copy(acc_sh.at[pl.ds(off, LK)], buf[k % GRP])
        pltpu.sync_copy(buf[k % GRP], out_hbm.at[pl.ds(c * NPAD + off, LK)])


RB = 512
SUB = RB // 128
NGRID = NPAD // RB


def _dinv_diag(d0_b, d1_b, k):
    dinv = lax.rsqrt(d0_b[0, k] + d1_b[0, k]).reshape(1, 128)
    ir = lax.broadcasted_iota(jnp.int32, (128, 128), 0)
    ic = lax.broadcasted_iota(jnp.int32, (128, 128), 1)
    return jnp.where(ir == ic, jnp.broadcast_to(dinv, (128, 128)), 0.0)


def _mm_body(x_b, w_b, y_b):
    y_b[...] = jnp.dot(x_b[...], w_b[...], preferred_element_type=jnp.float32)


def _scale_body(xw_b, d0_b, d1_b, y_b):
    for k in range(SUB):
        dm = _dinv_diag(d0_b, d1_b, k)
        sl = pl.ds(k * 128, 128)
        y_b[sl, :] = jnp.dot(dm, xw_b[sl, :], preferred_element_type=jnp.float32)


def _tc2_body(a0_b, a1_b, d0_b, d1_b, b1_b, w_b, y_b, h_ref):
    for k in range(SUB):
        dm = _dinv_diag(d0_b, d1_b, k)
        sl = pl.ds(k * 128, 128)
        agg = a0_b[sl, :] + a1_b[sl, :]
        h = jnp.dot(dm, agg, preferred_element_type=jnp.float32) + b1_b[...]
        h_ref[sl, :] = jnp.maximum(h, 0.0)
    hw = jnp.dot(h_ref[...], w_b[...], preferred_element_type=jnp.float32)
    for k in range(SUB):
        dm = _dinv_diag(d0_b, d1_b, k)
        hwk = hw[k * 128:(k + 1) * 128, :]
        y_b[pl.ds(k * 128, 128), :] = jnp.dot(
            dm, hwk, preferred_element_type=jnp.float32)


def _tc3_body(a0_b, a1_b, d0_b, d1_b, b2_b, y_b):
    for k in range(SUB):
        dm = _dinv_diag(d0_b, d1_b, k)
        sl = pl.ds(k * 128, 128)
        agg = a0_b[sl, :] + a1_b[sl, :]
        y_b[sl, :] = (jnp.dot(dm, agg, preferred_element_type=jnp.float32)
                      + b2_b[...])


_ROWBLK = pl.BlockSpec((RB, D), lambda i: (i, 0))
_FULLW = pl.BlockSpec((D, D), lambda i: (0, 0))
_DEGBLK = pl.BlockSpec((1, SUB, 128), lambda i: (i, 0, 0))
_BIAS = pl.BlockSpec((1, D), lambda i: (0, 0))
_OUTBLK = jax.ShapeDtypeStruct((NPAD, D), jnp.float32)

_mm = pl.pallas_call(
    _mm_body, grid=(NGRID,),
    in_specs=[_ROWBLK, _FULLW],
    out_specs=_ROWBLK, out_shape=_OUTBLK)

_scale = pl.pallas_call(
    _scale_body, grid=(NGRID,),
    in_specs=[_ROWBLK, _DEGBLK, _DEGBLK],
    out_specs=_ROWBLK, out_shape=_OUTBLK)

_tc2 = pl.pallas_call(
    _tc2_body, grid=(NGRID,),
    in_specs=[_ROWBLK, _ROWBLK, _DEGBLK, _DEGBLK, _BIAS, _FULLW],
    out_specs=_ROWBLK, out_shape=_OUTBLK,
    scratch_shapes=[pltpu.VMEM((RB, D), jnp.float32)])

_tc3 = pl.pallas_call(
    _tc3_body, grid=(NGRID,),
    in_specs=[_ROWBLK, _ROWBLK, _DEGBLK, _DEGBLK, _BIAS],
    out_specs=_ROWBLK, out_shape=_OUTBLK)


@jax.jit
def _run(x, edge_index, W1, b1, W2, b2):
    row = edge_index[0].astype(jnp.int32)
    col = edge_index[1].astype(jnp.int32)
    loop = jnp.arange(N, dtype=jnp.int32)
    npad_e = EPAD - (row.shape[0] + N)
    row = jnp.concatenate([row, loop, jnp.zeros((npad_e,), jnp.int32)])
    pad_col = N + (jnp.arange(npad_e, dtype=jnp.int32) % (NPAD - N))
    col = jnp.concatenate([col, loop, pad_col])
    packed = (row | (col << 15)).reshape(NW * CHUNKS, LK)

    xp = jnp.zeros((NPAD, D), x.dtype).at[:N].set(x)
    z1 = jnp.zeros((NPAD,), jnp.float32)
    z2 = jnp.zeros((NPAD, D), jnp.float32)

    xw1 = _mm(xp, W1)
    deg = _deg(packed, z1).reshape(NC, NGRID, SUB, 128)
    d0, d1 = deg[0], deg[1]

    y1 = _scale(xw1, d0, d1)
    a = _agg(y1, packed, z2).reshape(NC, NPAD, D)
    y2 = _tc2(a[0], a[1], d0, d1, b1.reshape(1, D), W2)
    a = _agg(y2, packed, z2).reshape(NC, NPAD, D)
    out = _tc3(a[0], a[1], d0, d1, b2.reshape(1, D))
    return out[:N]


def kernel(x, edge_index, W1, b1, W2, b2):
    return _run(x, edge_index, W1, b1, W2, b2)


# speedup vs baseline: 1.1328x; 1.0042x over previous
import functools

import jax
import jax.numpy as jnp
from jax import lax
from jax.experimental import pallas as pl
from jax.experimental.pallas import tpu as pltpu
from jax.experimental.pallas import tpu_sc as plsc

N = 10000
D = 128
NPAD = 10240
NBLK = NPAD // 128
NC = 2
NS = 16
NW = NC * NS
LK = 128
GRP = 2
CHUNKS = 82
GROUPS = CHUNKS // GRP
EPAD = NW * CHUNKS * LK
ROWS_PER_TILE = NPAD // NS


def _mesh():
    return plsc.VectorSubcoreMesh(core_axis_name="c", subcore_axis_name="s")


def _split_packed(pk, k, ri, ci):
    for m in range(LK // 16):
        sl = pl.ds(m * 16, 16)
        p = pk[k, sl]
        ri[sl] = p & 0x7FFF
        ci[sl] = lax.shift_right_logical(p, 15)


@functools.partial(
    pl.kernel,
    out_type=jax.ShapeDtypeStruct((NC * NPAD,), jnp.float32),
    mesh=_mesh(),
    scratch_types=[
        pltpu.VMEM((GRP, LK), jnp.int32),
        pltpu.VMEM((LK,), jnp.int32),
        pltpu.VMEM((LK,), jnp.int32),
        pltpu.VMEM((LK,), jnp.float32),
        pltpu.VMEM((ROWS_PER_TILE,), jnp.float32),
        pltpu.VMEM_SHARED((NPAD,), jnp.float32),
    ],
)
def _deg(pkb_hbm, zeros1_hbm, out_hbm, pk_v, ri_v, ci_v, ones_v, rb_v, deg_sh):
    c = lax.axis_index("c")
    s = lax.axis_index("s")
    wid = c * NS + s
    base = s * ROWS_PER_TILE
    pltpu.sync_copy(zeros1_hbm.at[pl.ds(base, ROWS_PER_TILE)],
                    deg_sh.at[pl.ds(base, ROWS_PER_TILE)])
    for i in range(LK // 16):
        ones_v[pl.ds(i * 16, 16)] = jnp.full((16,), 1.0, jnp.float32)
    plsc.subcore_barrier()

    def body(jj, carry):
        pltpu.sync_copy(pkb_hbm.at[pl.ds((jj * NW + wid) * GRP, GRP)], pk_v)
        for k in range(GRP):
            _split_packed(pk_v, k, ri_v, ci_v)
            pltpu.sync_copy(ones_v, deg_sh.at[ci_v], add=True)
        return carry

    lax.fori_loop(0, GROUPS, body, 0)
    plsc.subcore_barrier()
    pltpu.sync_copy(deg_sh.at[pl.ds(base, ROWS_PER_TILE)], rb_v)
    pltpu.sync_copy(rb_v, out_hbm.at[pl.ds(c * NPAD + base, ROWS_PER_TILE)])


@functools.partial(
    pl.kernel,
    out_type=jax.ShapeDtypeStruct((NC * NPAD, D), jnp.float32),
    mesh=_mesh(),
    scratch_types=[
        pltpu.VMEM((GRP, LK), jnp.int32),
        [pltpu.VMEM((LK,), jnp.int32) for _ in range(GRP)],
        [pltpu.VMEM((LK,), jnp.int32) for _ in range(GRP)],
        [pltpu.VMEM((LK, D), jnp.float32) for _ in range(GRP)],
        [pltpu.SemaphoreType.DMA for _ in range(GRP)],
        [pltpu.SemaphoreType.DMA for _ in range(GRP)],
        pltpu.VMEM_SHARED((NPAD, D), jnp.float32),
    ],
)
def _agg(y_hbm, pkb_hbm, zeros2_hbm, out_hbm,
         pk_v, ri, ci, buf, sg, ss, acc_sh):
    c = lax.axis_index("c")
    s = lax.axis_index("s")
    wid = c * NS + s
    base = s * ROWS_PER_TILE
    pltpu.sync_copy(zeros2_hbm.at[pl.ds(base, ROWS_PER_TILE)],
                    acc_sh.at[pl.ds(base, ROWS_PER_TILE)])
    plsc.subcore_barrier()

    def body(jj, carry):
        @pl.when(jj > 0)
        def _():
            for k in range(GRP):
                pltpu.make_async_copy(y_hbm.at[pl.ds(0, LK)], buf[k],
                                      ss[k]).wait()
        pltpu.sync_copy(pkb_hbm.at[pl.ds((jj * NW + wid) * GRP, GRP)], pk_v)
        cps = []
        for k in range(GRP):
            _split_packed(pk_v, k, ri[k], ci[k])
            cps.append(pltpu.async_copy(y_hbm.at[ri[k]], buf[k], sg[k]))
        for k in range(GRP):
            cps[k].wait()
            pltpu.async_copy(buf[k], acc_sh.at[ci[k]], ss[k], add=True)
        return carry

    lax.fori_loop(0, GROUPS, body, 0)
    for k in range(GRP):
        pltpu.make_async_copy(y_hbm.at[pl.ds(0, LK)], buf[k], ss[k]).wait()
    plsc.subcore_barrier()
    for k in range(ROWS_PER_TILE // LK):
        off = base + k * LK
        pltpu.sync_copy(acc_sh.at[pl.ds(off, LK)], buf[k % GRP])
        pltpu.sync_copy(buf[k % GRP], out_hbm.at[pl.ds(c * NPAD + off, LK)])


RB = 512
SUB = RB // 128
NGRID = NPAD // RB


def _dinv_diag(d0_b, d1_b, k):
    dinv = lax.rsqrt(d0_b[0, k] + d1_b[0, k]).reshape(1, 128)
    ir = lax.broadcasted_iota(jnp.int32, (128, 128), 0)
    ic = lax.broadcasted_iota(jnp.int32, (128, 128), 1)
    return jnp.where(ir == ic, jnp.broadcast_to(dinv, (128, 128)), 0.0)


def _mm_body(x_b, w_b, y_b):
    y_b[...] = jnp.dot(x_b[...], w_b[...], preferred_element_type=jnp.float32)


def _scale_body(xw_b, d0_b, d1_b, y_b):
    for k in range(SUB):
        dm = _dinv_diag(d0_b, d1_b, k)
        sl = pl.ds(k * 128, 128)
        y_b[sl, :] = jnp.dot(dm, xw_b[sl, :], preferred_element_type=jnp.float32)


def _tc2_body(a0_b, a1_b, d0_b, d1_b, b1_b, w_b, y_b, h_ref):
    for k in range(SUB):
        dm = _dinv_diag(d0_b, d1_b, k)
        sl = pl.ds(k * 128, 128)
        agg = a0_b[sl, :] + a1_b[sl, :]
        h = jnp.dot(dm, agg, preferred_element_type=jnp.float32) + b1_b[...]
        h_ref[sl, :] = jnp.maximum(h, 0.0)
    hw = jnp.dot(h_ref[...], w_b[...], preferred_element_type=jnp.float32)
    for k in range(SUB):
        dm = _dinv_diag(d0_b, d1_b, k)
        hwk = hw[k * 128:(k + 1) * 128, :]
        y_b[pl.ds(k * 128, 128), :] = jnp.dot(
            dm, hwk, preferred_element_type=jnp.float32)


def _tc3_body(a0_b, a1_b, d0_b, d1_b, b2_b, y_b):
    for k in range(SUB):
        dm = _dinv_diag(d0_b, d1_b, k)
        sl = pl.ds(k * 128, 128)
        agg = a0_b[sl, :] + a1_b[sl, :]
        y_b[sl, :] = (jnp.dot(dm, agg, preferred_element_type=jnp.float32)
                      + b2_b[...])


_ROWBLK = pl.BlockSpec((RB, D), lambda i: (i, 0))
_FULLW = pl.BlockSpec((D, D), lambda i: (0, 0))
_DEGBLK = pl.BlockSpec((1, SUB, 128), lambda i: (i, 0, 0))
_BIAS = pl.BlockSpec((1, D), lambda i: (0, 0))
_OUTBLK = jax.ShapeDtypeStruct((NPAD, D), jnp.float32)

_mm = pl.pallas_call(
    _mm_body, grid=(NGRID,),
    in_specs=[_ROWBLK, _FULLW],
    out_specs=_ROWBLK, out_shape=_OUTBLK)

_scale = pl.pallas_call(
    _scale_body, grid=(NGRID,),
    in_specs=[_ROWBLK, _DEGBLK, _DEGBLK],
    out_specs=_ROWBLK, out_shape=_OUTBLK)

_tc2 = pl.pallas_call(
    _tc2_body, grid=(NGRID,),
    in_specs=[_ROWBLK, _ROWBLK, _DEGBLK, _DEGBLK, _BIAS, _FULLW],
    out_specs=_ROWBLK, out_shape=_OUTBLK,
    scratch_shapes=[pltpu.VMEM((RB, D), jnp.float32)])

_tc3 = pl.pallas_call(
    _tc3_body, grid=(NGRID,),
    in_specs=[_ROWBLK, _ROWBLK, _DEGBLK, _DEGBLK, _BIAS],
    out_specs=_ROWBLK, out_shape=_OUTBLK)


@jax.jit
def _run(x, edge_index, W1, b1, W2, b2):
    row = edge_index[0].astype(jnp.int32)
    col = edge_index[1].astype(jnp.int32)
    loop = jnp.arange(N, dtype=jnp.int32)
    npad_e = EPAD - (row.shape[0] + N)
    row = jnp.concatenate([row, loop, jnp.zeros((npad_e,), jnp.int32)])
    pad_col = N + (jnp.arange(npad_e, dtype=jnp.int32) % (NPAD - N))
    col = jnp.concatenate([col, loop, pad_col])
    packed = (row | (col << 15)).reshape(NW * CHUNKS, LK)

    xp = jnp.zeros((NPAD, D), x.dtype).at[:N].set(x)
    z1 = jnp.zeros((NPAD,), jnp.float32)
    z2 = jnp.zeros((NPAD, D), jnp.float32)

    xw1 = _mm(xp, W1)
    deg = _deg(packed, z1).reshape(NC, NGRID, SUB, 128)
    d0, d1 = deg[0], deg[1]

    y1 = _scale(xw1, d0, d1)
    a = _agg(y1, packed, z2).reshape(NC, NPAD, D)
    y2 = _tc2(a[0], a[1], d0, d1, b1.reshape(1, D), W2)
    a = _agg(y2, packed, z2).reshape(NC, NPAD, D)
    out = _tc3(a[0], a[1], d0, d1, b2.reshape(1, D))
    return out[:N]


def kernel(x, edge_index, W1, b1, W2, b2):
    return _run(x, edge_index, W1, b1, W2, b2)
